# baseline (speedup 1.0000x reference)
import functools

import jax
import jax.numpy as jnp
from jax import lax
from jax.experimental import pallas as pl
from jax.experimental.pallas import tpu as pltpu
from jax.experimental.pallas import tpu_sc as plsc

E = 8
D = 1024
DFF = 1024
T = 2048
BT = 256
NB = 2 * T // BT + E
NP = NB * BT
NW = 32
CH = 256


_TDIMS = (((1,), (1,)), ((), ()))


def _router_body(x_ref, gate_ref, bias_ref, w3_ref,
                 pos_ref, w0_ref, w1_ref, blk_ref, nact_ref, w3b_ref):
    w3b_ref[...] = w3_ref[...].astype(jnp.bfloat16)

    @pl.when(pl.program_id(0) == 0)
    def _routing():
        _router_compute(x_ref, gate_ref, bias_ref,
                        pos_ref, w0_ref, w1_ref, blk_ref, nact_ref)


def _router_compute(x_ref, gate_ref, bias_ref,
                    pos_ref, w0_ref, w1_ref, blk_ref, nact_ref):
    x = x_ref[...]
    logits = lax.dot_general(x, gate_ref[...], _TDIMS,
                             preferred_element_type=jnp.float32)
    scores = jax.nn.sigmoid(logits)
    sfc = scores + bias_ref[...]
    lane = lax.broadcasted_iota(jnp.int32, (T, E), 1)
    m1 = jnp.max(sfc, axis=1, keepdims=True)
    i1 = jnp.min(jnp.where(sfc == m1, lane, E), axis=1, keepdims=True)
    sfc2 = jnp.where(lane == i1, -jnp.inf, sfc)
    m2 = jnp.max(sfc2, axis=1, keepdims=True)
    i2 = jnp.min(jnp.where(sfc2 == m2, lane, E), axis=1, keepdims=True)
    oh0 = (lane == i1).astype(jnp.float32)
    oh1 = (lane == i2).astype(jnp.float32)
    s0 = jnp.sum(oh0 * scores, axis=1, keepdims=True)
    s1 = jnp.sum(oh1 * scores, axis=1, keepdims=True)
    denom = s0 + s1
    w0_ref[...] = jnp.broadcast_to(s0 / denom, (T, 16))
    w1_ref[...] = jnp.broadcast_to(s1 / denom, (T, 16))

    counts0 = jnp.sum(oh0, axis=0, keepdims=True)
    counts1 = jnp.sum(oh1, axis=0, keepdims=True)
    counts = counts0 + counts1
    padded = jnp.ceil(counts * (1.0 / BT)) * BT
    triu = (lax.broadcasted_iota(jnp.int32, (E, E), 0)
            < lax.broadcasted_iota(jnp.int32, (E, E), 1)).astype(jnp.float32)
    starts = jnp.dot(padded, triu, preferred_element_type=jnp.float32)

    tris = (lax.broadcasted_iota(jnp.int32, (CH, CH), 0)
            > lax.broadcasted_iota(jnp.int32, (CH, CH), 1)).astype(jnp.float32)

    def emit_pos(oh, base, row0):
        carry = jnp.zeros((1, E), jnp.float32)
        for c in range(T // CH):
            ohc = oh[c * CH:(c + 1) * CH]
            before = carry + jnp.dot(tris, ohc,
                                     preferred_element_type=jnp.float32)
            pos = jnp.sum(ohc * (base + before), axis=1, keepdims=True)
            pos_ref[row0 + c * CH:row0 + (c + 1) * CH, :] = pos.astype(jnp.int32)
            carry = carry + jnp.sum(ohc, axis=0, keepdims=True)

    emit_pos(oh0, starts, 0)
    emit_pos(oh1, starts + counts0, T)

    bval = lax.broadcasted_iota(jnp.int32, (NB, E), 0).astype(jnp.float32) * BT
    blk = jnp.sum((bval >= starts).astype(jnp.float32), axis=1,
                  keepdims=True) - 1.0
    blk_ref[...] = blk.astype(jnp.int32)
    nact_ref[...] = (jnp.sum(padded, axis=1, keepdims=True)
                     * (1.0 / BT)).astype(jnp.int32)


def _router_call(x, gate_w, bias, w3, interpret=False):
    full = lambda e: (0, 0)
    outs = pl.pallas_call(
        _router_body,
        grid=(E,),
        in_specs=[
            pl.BlockSpec((T, D), full),
            pl.BlockSpec((E, D), full),
            pl.BlockSpec((1, E), full),
            pl.BlockSpec((DFF, D), lambda e: (e, 0)),
        ],
        out_specs=[
            pl.BlockSpec((2 * T, 1), full),
            pl.BlockSpec((T, 16), full),
            pl.BlockSpec((T, 16), full),
            pl.BlockSpec((NB, 1), full),
            pl.BlockSpec((1, 1), full),
            pl.BlockSpec((DFF, D), lambda e: (e, 0)),
        ],
        out_shape=(
            jax.ShapeDtypeStruct((2 * T, 1), jnp.int32),
            jax.ShapeDtypeStruct((T, 16), jnp.float32),
            jax.ShapeDtypeStruct((T, 16), jnp.float32),
            jax.ShapeDtypeStruct((NB, 1), jnp.int32),
            jax.ShapeDtypeStruct((1, 1), jnp.int32),
            jax.ShapeDtypeStruct((E * DFF, D), jnp.bfloat16),
        ),
        interpret=interpret,
    )(x, gate_w, bias, w3.reshape(E * DFF, D))
    return outs[:5] + (outs[5].reshape(E, DFF, D),)


def _wcast_body(w1_ref, w2_ref, o1_ref, o2_ref):
    o1_ref[...] = pltpu.bitcast(w1_ref[...].astype(jnp.bfloat16), jnp.int32)
    o2_ref[...] = pltpu.bitcast(w2_ref[...].astype(jnp.bfloat16), jnp.int32)


def _wcast_call(w1, w2, interpret=False):
    spec = pl.BlockSpec((DFF // 2, D), lambda s: (s, 0))
    ospec = pl.BlockSpec((DFF // 4, D), lambda s: (s, 0))
    outs = pl.pallas_call(
        _wcast_body,
        grid=(2 * E,),
        in_specs=[spec, spec],
        out_specs=[ospec, ospec],
        out_shape=[
            jax.ShapeDtypeStruct((E * DFF // 2, D), jnp.int32),
            jax.ShapeDtypeStruct((E * D // 2, DFF), jnp.int32),
        ],
        interpret=interpret,
    )(w1.reshape(E * DFF, D), w2.reshape(E * D, DFF))
    return (outs[0].reshape(E, DFF // 2, D), outs[1].reshape(E, D // 2, DFF))


def _gmm_body(blk_ref, nact_ref, xd_ref, w1_ref, w3_ref, w2_ref, out_ref):
    b = pl.program_id(0)

    @pl.when(b < nact_ref[0])
    def _():
        xb = xd_ref[...].astype(jnp.bfloat16)
        t1 = lax.dot_general(xb, pltpu.bitcast(w1_ref[0], jnp.bfloat16),
                             _TDIMS, preferred_element_type=jnp.float32)
        t3 = lax.dot_general(xb, w3_ref[0], _TDIMS,
                             preferred_element_type=jnp.float32)
        h = (t1 * jax.nn.sigmoid(t1) * t3).astype(jnp.bfloat16)
        out_ref[...] = lax.dot_general(h, pltpu.bitcast(w2_ref[0], jnp.bfloat16),
                                       _TDIMS, preferred_element_type=jnp.float32)


def _gmm_call(blk, nact, xdisp, w1t, w3t, w2t, interpret=False):
    act = lambda b, na: jnp.where(b < na[0], b, NB - 1)
    grid_spec = pltpu.PrefetchScalarGridSpec(
        num_scalar_prefetch=2,
        grid=(NB,),
        in_specs=[
            pl.BlockSpec((BT, D), lambda b, blk, na: (act(b, na), 0)),
            pl.BlockSpec((1, DFF // 2, D), lambda b, blk, na: (blk[b], 0, 0)),
            pl.BlockSpec((1, DFF, D), lambda b, blk, na: (blk[b], 0, 0)),
            pl.BlockSpec((1, D // 2, DFF), lambda b, blk, na: (blk[b], 0, 0)),
        ],
        out_specs=pl.BlockSpec((BT, D), lambda b, blk, na: (act(b, na), 0)),
    )
    return pl.pallas_call(
        _gmm_body,
        grid_spec=grid_spec,
        out_shape=jax.ShapeDtypeStruct((NP, D), jnp.float32),
        interpret=interpret,
    )(blk, nact, xdisp, w1t, w3t, w2t)


@functools.cache
def _sc_mesh():
    return plsc.VectorSubcoreMesh(core_axis_name="c", subcore_axis_name="s")


@functools.cache
def _dispatch_kernel():
    @functools.partial(
        pl.kernel,
        out_type=jax.ShapeDtypeStruct((NP, D), jnp.float32),
        mesh=_sc_mesh(),
        scratch_types=[
            pltpu.VMEM((64,), jnp.int32),
            pltpu.VMEM((64, D), jnp.float32),
            pltpu.SemaphoreType.DMA,
        ],
    )
    def _dispatch_call(x_hbm, pos_hbm, xd_hbm, idx_v, buf, sem):
        wid = lax.axis_index("s") * 2 + lax.axis_index("c")
        for c in range(2):
            p0 = wid * 128 + c * 64
            t0 = lax.rem(p0, T)
            pltpu.sync_copy(pos_hbm.at[pl.ds(p0, 64)], idx_v)
            pltpu.sync_copy(x_hbm.at[pl.ds(t0, 64)], buf)
            pltpu.async_copy(buf, xd_hbm.at[idx_v], sem).wait()

    return _dispatch_call


@functools.cache
def _combine_kernel():
    @functools.partial(
        pl.kernel,
        out_type=jax.ShapeDtypeStruct((T, D), jnp.float32),
        mesh=_sc_mesh(),
        scratch_types=[
            pltpu.VMEM((32,), jnp.int32),
            pltpu.VMEM((32,), jnp.int32),
            pltpu.VMEM((32, D), jnp.float32),
            pltpu.VMEM((32, D), jnp.float32),
            pltpu.VMEM((32, 16), jnp.float32),
            pltpu.VMEM((32, 16), jnp.float32),
            pltpu.SemaphoreType.DMA,
        ],
    )
    def _combine_call(y_hbm, pos_hbm, w0_hbm, w1_hbm, out_hbm,
                      idx0, idx1, yb0, yb1, w0s, w1s, sem):
        wid = lax.axis_index("s") * 2 + lax.axis_index("c")
        for c in range(2):
            tb = wid * 64 + c * 32
            pltpu.sync_copy(pos_hbm.at[pl.ds(tb, 32)], idx0)
            pltpu.sync_copy(pos_hbm.at[pl.ds(T + tb, 32)], idx1)
            pltpu.sync_copy(w0_hbm.at[pl.ds(tb, 32)], w0s)
            pltpu.sync_copy(w1_hbm.at[pl.ds(tb, 32)], w1s)
            pltpu.async_copy(y_hbm.at[idx0], yb0, sem).wait()
            pltpu.async_copy(y_hbm.at[idx1], yb1, sem).wait()

            def row(i, _):
                a = w0s[i, :]
                b = w1s[i, :]
                for j in range(D // 16):
                    sl = pl.ds(j * 16, 16)
                    yb0[i, sl] = a * yb0[i, sl] + b * yb1[i, sl]
                return 0

            lax.fori_loop(0, 32, row, 0)
            pltpu.sync_copy(yb0, out_hbm.at[pl.ds(tb, 32)])

    return _combine_call


def kernel(hidden_states, gate_w, e_score_correction_bias, w1, w3, w2):
    orig_shape = hidden_states.shape
    x = hidden_states.reshape(T, D)
    bias = e_score_correction_bias.reshape(1, E)
    w1b, w2b = _wcast_call(w1, w2)

    pos, w0, w1n, blk, nact, w3b = _router_call(x, gate_w, bias, w3)
    pos = pos.reshape(2 * T)
    xdisp = _dispatch_kernel()(x, pos)
    ydisp = _gmm_call(blk.reshape(NB), nact.reshape(1), xdisp, w1b, w3b, w2b)
    out = _combine_kernel()(ydisp, pos, w0, w1n)
    return out.reshape(orig_shape)


# speedup vs baseline: 1.0368x; 1.0368x over previous
import functools

import jax
import jax.numpy as jnp
from jax import lax
from jax.experimental import pallas as pl
from jax.experimental.pallas import tpu as pltpu
from jax.experimental.pallas import tpu_sc as plsc

E = 8
D = 1024
DFF = 1024
T = 2048
BT = 512
NB = 2 * T // BT + E
NP = NB * BT
NW = 32
CH = 256


_TDIMS = (((1,), (1,)), ((), ()))


def _router_body(x_ref, gate_ref, bias_ref, w3_ref,
                 pos_ref, w0_ref, w1_ref, blk_ref, nact_ref, w3b_ref):
    w3b_ref[...] = w3_ref[...].astype(jnp.bfloat16)

    @pl.when(pl.program_id(0) == 0)
    def _routing():
        _router_compute(x_ref, gate_ref, bias_ref,
                        pos_ref, w0_ref, w1_ref, blk_ref, nact_ref)


def _router_compute(x_ref, gate_ref, bias_ref,
                    pos_ref, w0_ref, w1_ref, blk_ref, nact_ref):
    x = x_ref[...]
    logits = lax.dot_general(x, gate_ref[...], _TDIMS,
                             preferred_element_type=jnp.float32)
    scores = jax.nn.sigmoid(logits)
    sfc = scores + bias_ref[...]
    lane = lax.broadcasted_iota(jnp.int32, (T, E), 1)
    m1 = jnp.max(sfc, axis=1, keepdims=True)
    i1 = jnp.min(jnp.where(sfc == m1, lane, E), axis=1, keepdims=True)
    sfc2 = jnp.where(lane == i1, -jnp.inf, sfc)
    m2 = jnp.max(sfc2, axis=1, keepdims=True)
    i2 = jnp.min(jnp.where(sfc2 == m2, lane, E), axis=1, keepdims=True)
    oh0 = (lane == i1).astype(jnp.float32)
    oh1 = (lane == i2).astype(jnp.float32)
    s0 = jnp.sum(oh0 * scores, axis=1, keepdims=True)
    s1 = jnp.sum(oh1 * scores, axis=1, keepdims=True)
    denom = s0 + s1
    w0_ref[...] = jnp.broadcast_to(s0 / denom, (T, 16))
    w1_ref[...] = jnp.broadcast_to(s1 / denom, (T, 16))

    counts0 = jnp.sum(oh0, axis=0, keepdims=True)
    counts1 = jnp.sum(oh1, axis=0, keepdims=True)
    counts = counts0 + counts1
    padded = jnp.ceil(counts * (1.0 / BT)) * BT
    triu = (lax.broadcasted_iota(jnp.int32, (E, E), 0)
            < lax.broadcasted_iota(jnp.int32, (E, E), 1)).astype(jnp.float32)
    starts = jnp.dot(padded, triu, preferred_element_type=jnp.float32)

    tris = (lax.broadcasted_iota(jnp.int32, (CH, CH), 0)
            > lax.broadcasted_iota(jnp.int32, (CH, CH), 1)).astype(jnp.float32)

    def emit_pos(oh, base, row0):
        carry = jnp.zeros((1, E), jnp.float32)
        for c in range(T // CH):
            ohc = oh[c * CH:(c + 1) * CH]
            before = carry + jnp.dot(tris, ohc,
                                     preferred_element_type=jnp.float32)
            pos = jnp.sum(ohc * (base + before), axis=1, keepdims=True)
            pos_ref[row0 + c * CH:row0 + (c + 1) * CH, :] = pos.astype(jnp.int32)
            carry = carry + jnp.sum(ohc, axis=0, keepdims=True)

    emit_pos(oh0, starts, 0)
    emit_pos(oh1, starts + counts0, T)

    bval = lax.broadcasted_iota(jnp.int32, (NB, E), 0).astype(jnp.float32) * BT
    blk = jnp.sum((bval >= starts).astype(jnp.float32), axis=1,
                  keepdims=True) - 1.0
    blk_ref[...] = blk.astype(jnp.int32)
    nact_ref[...] = (jnp.sum(padded, axis=1, keepdims=True)
                     * (1.0 / BT)).astype(jnp.int32)


def _router_call(x, gate_w, bias, w3, interpret=False):
    full = lambda e: (0, 0)
    outs = pl.pallas_call(
        _router_body,
        grid=(E,),
        in_specs=[
            pl.BlockSpec((T, D), full),
            pl.BlockSpec((E, D), full),
            pl.BlockSpec((1, E), full),
            pl.BlockSpec((DFF, D), lambda e: (e, 0)),
        ],
        out_specs=[
            pl.BlockSpec((2 * T, 1), full),
            pl.BlockSpec((T, 16), full),
            pl.BlockSpec((T, 16), full),
            pl.BlockSpec((NB, 1), full),
            pl.BlockSpec((1, 1), full),
            pl.BlockSpec((DFF, D), lambda e: (e, 0)),
        ],
        out_shape=(
            jax.ShapeDtypeStruct((2 * T, 1), jnp.int32),
            jax.ShapeDtypeStruct((T, 16), jnp.float32),
            jax.ShapeDtypeStruct((T, 16), jnp.float32),
            jax.ShapeDtypeStruct((NB, 1), jnp.int32),
            jax.ShapeDtypeStruct((1, 1), jnp.int32),
            jax.ShapeDtypeStruct((E * DFF, D), jnp.bfloat16),
        ),
        interpret=interpret,
    )(x, gate_w, bias, w3.reshape(E * DFF, D))
    return outs[:5] + (outs[5].reshape(E, DFF, D),)


def _wcast_body(w1_ref, w2_ref, o1_ref, o2_ref):
    o1_ref[...] = pltpu.bitcast(w1_ref[...].astype(jnp.bfloat16), jnp.int32)
    o2_ref[...] = pltpu.bitcast(w2_ref[...].astype(jnp.bfloat16), jnp.int32)


def _wcast_call(w1, w2, interpret=False):
    spec = pl.BlockSpec((DFF // 2, D), lambda s: (s, 0))
    ospec = pl.BlockSpec((DFF // 4, D), lambda s: (s, 0))
    outs = pl.pallas_call(
        _wcast_body,
        grid=(2 * E,),
        in_specs=[spec, spec],
        out_specs=[ospec, ospec],
        out_shape=[
            jax.ShapeDtypeStruct((E * DFF // 2, D), jnp.int32),
            jax.ShapeDtypeStruct((E * D // 2, DFF), jnp.int32),
        ],
        interpret=interpret,
    )(w1.reshape(E * DFF, D), w2.reshape(E * D, DFF))
    return (outs[0].reshape(E, DFF // 2, D), outs[1].reshape(E, D // 2, DFF))


def _gmm_body(blk_ref, nact_ref, xd_ref, w1_ref, w3_ref, w2_ref, out_ref):
    b = pl.program_id(0)

    @pl.when(b < nact_ref[0])
    def _():
        xb = xd_ref[...].astype(jnp.bfloat16)
        t1 = lax.dot_general(xb, pltpu.bitcast(w1_ref[0], jnp.bfloat16),
                             _TDIMS, preferred_element_type=jnp.float32)
        t3 = lax.dot_general(xb, w3_ref[0], _TDIMS,
                             preferred_element_type=jnp.float32)
        h = (t1 * jax.nn.sigmoid(t1) * t3).astype(jnp.bfloat16)
        out_ref[...] = lax.dot_general(h, pltpu.bitcast(w2_ref[0], jnp.bfloat16),
                                       _TDIMS, preferred_element_type=jnp.float32)


def _gmm_call(blk, nact, xdisp, w1t, w3t, w2t, interpret=False):
    act = lambda b, na: jnp.where(b < na[0], b, NB - 1)
    grid_spec = pltpu.PrefetchScalarGridSpec(
        num_scalar_prefetch=2,
        grid=(NB,),
        in_specs=[
            pl.BlockSpec((BT, D), lambda b, blk, na: (act(b, na), 0)),
            pl.BlockSpec((1, DFF // 2, D), lambda b, blk, na: (blk[b], 0, 0)),
            pl.BlockSpec((1, DFF, D), lambda b, blk, na: (blk[b], 0, 0)),
            pl.BlockSpec((1, D // 2, DFF), lambda b, blk, na: (blk[b], 0, 0)),
        ],
        out_specs=pl.BlockSpec((BT, D), lambda b, blk, na: (act(b, na), 0)),
    )
    return pl.pallas_call(
        _gmm_body,
        grid_spec=grid_spec,
        out_shape=jax.ShapeDtypeStruct((NP, D), jnp.float32),
        interpret=interpret,
    )(blk, nact, xdisp, w1t, w3t, w2t)


@functools.cache
def _sc_mesh():
    return plsc.VectorSubcoreMesh(core_axis_name="c", subcore_axis_name="s")


@functools.cache
def _dispatch_kernel():
    @functools.partial(
        pl.kernel,
        out_type=jax.ShapeDtypeStruct((NP, D), jnp.float32),
        mesh=_sc_mesh(),
        scratch_types=[
            pltpu.VMEM((64,), jnp.int32),
            pltpu.VMEM((64, D), jnp.float32),
            pltpu.SemaphoreType.DMA,
        ],
    )
    def _dispatch_call(x_hbm, pos_hbm, xd_hbm, idx_v, buf, sem):
        wid = lax.axis_index("s") * 2 + lax.axis_index("c")
        for c in range(2):
            p0 = wid * 128 + c * 64
            t0 = lax.rem(p0, T)
            pltpu.sync_copy(pos_hbm.at[pl.ds(p0, 64)], idx_v)
            pltpu.sync_copy(x_hbm.at[pl.ds(t0, 64)], buf)
            pltpu.async_copy(buf, xd_hbm.at[idx_v], sem).wait()

    return _dispatch_call


@functools.cache
def _combine_kernel():
    @functools.partial(
        pl.kernel,
        out_type=jax.ShapeDtypeStruct((T, D), jnp.float32),
        mesh=_sc_mesh(),
        scratch_types=[
            pltpu.VMEM((32,), jnp.int32),
            pltpu.VMEM((32,), jnp.int32),
            pltpu.VMEM((32, D), jnp.float32),
            pltpu.VMEM((32, D), jnp.float32),
            pltpu.VMEM((32, 16), jnp.float32),
            pltpu.VMEM((32, 16), jnp.float32),
            pltpu.SemaphoreType.DMA,
        ],
    )
    def _combine_call(y_hbm, pos_hbm, w0_hbm, w1_hbm, out_hbm,
                      idx0, idx1, yb0, yb1, w0s, w1s, sem):
        wid = lax.axis_index("s") * 2 + lax.axis_index("c")
        for c in range(2):
            tb = wid * 64 + c * 32
            pltpu.sync_copy(pos_hbm.at[pl.ds(tb, 32)], idx0)
            pltpu.sync_copy(pos_hbm.at[pl.ds(T + tb, 32)], idx1)
            pltpu.sync_copy(w0_hbm.at[pl.ds(tb, 32)], w0s)
            pltpu.sync_copy(w1_hbm.at[pl.ds(tb, 32)], w1s)
            pltpu.async_copy(y_hbm.at[idx0], yb0, sem).wait()
            pltpu.async_copy(y_hbm.at[idx1], yb1, sem).wait()

            def row(i, _):
                a = w0s[i, :]
                b = w1s[i, :]
                for j in range(D // 16):
                    sl = pl.ds(j * 16, 16)
                    yb0[i, sl] = a * yb0[i, sl] + b * yb1[i, sl]
                return 0

            lax.fori_loop(0, 32, row, 0)
            pltpu.sync_copy(yb0, out_hbm.at[pl.ds(tb, 32)])

    return _combine_call


def kernel(hidden_states, gate_w, e_score_correction_bias, w1, w3, w2):
    orig_shape = hidden_states.shape
    x = hidden_states.reshape(T, D)
    bias = e_score_correction_bias.reshape(1, E)
    w1b, w2b = _wcast_call(w1, w2)

    pos, w0, w1n, blk, nact, w3b = _router_call(x, gate_w, bias, w3)
    pos = pos.reshape(2 * T)
    xdisp = _dispatch_kernel()(x, pos)
    ydisp = _gmm_call(blk.reshape(NB), nact.reshape(1), xdisp, w1b, w3b, w2b)
    out = _combine_kernel()(ydisp, pos, w0, w1n)
    return out.reshape(orig_shape)
